# baseline (speedup 1.0000x reference)
import functools

import jax
import jax.numpy as jnp
from jax import lax
from jax.experimental import pallas as pl
from jax.experimental.pallas import tpu as pltpu
from jax.experimental.pallas import tpu_sc as plsc

NUM_EMBEDDINGS = 1000000
EMBEDDING_DIM = 64
BATCH = 4096
SEQ = 200

NC = 2
NS = 16
NW = NC * NS
B = BATCH * SEQ
CHUNK = 128
B_PER_W = B // NW
N_CHUNKS = B_PER_W // CHUNK


def _make_gather():
    mesh = plsc.VectorSubcoreMesh(core_axis_name="c", subcore_axis_name="s")

    @functools.partial(
        pl.kernel,
        out_type=jax.ShapeDtypeStruct((B, EMBEDDING_DIM), jnp.float32),
        mesh=mesh,
        scratch_types=[
            pltpu.VMEM((N_CHUNKS, CHUNK), jnp.int32),
            pltpu.VMEM((CHUNK, EMBEDDING_DIM), jnp.float32),
            pltpu.SemaphoreType.DMA,
        ],
        compiler_params=pltpu.CompilerParams(use_tc_tiling_on_sc=False),
    )
    def gather_kernel(idx_hbm, table_hbm, out_hbm, idx_v, rows_v, gsem):
        wid = lax.axis_index("s") * NC + lax.axis_index("c")
        base = wid * B_PER_W
        pltpu.sync_copy(idx_hbm.at[wid], idx_v)

        def body(j, carry):
            pltpu.async_copy(table_hbm.at[idx_v.at[j]], rows_v, gsem).wait()
            pltpu.sync_copy(rows_v, out_hbm.at[pl.ds(base + j * CHUNK, CHUNK)])
            return carry

        lax.fori_loop(0, N_CHUNKS, body, 0)

    return gather_kernel


_gather = _make_gather()


def kernel(input_ids, weight):
    idx = input_ids.astype(jnp.int32).reshape(NW, N_CHUNKS, CHUNK)
    out = _gather(idx, weight)
    return out.reshape(BATCH, SEQ, EMBEDDING_DIM)


# speedup vs baseline: 1.1166x; 1.1166x over previous
import functools

import jax
import jax.numpy as jnp
from jax import lax
from jax.experimental import pallas as pl
from jax.experimental.pallas import tpu as pltpu
from jax.experimental.pallas import tpu_sc as plsc

NUM_EMBEDDINGS = 1000000
EMBEDDING_DIM = 64
BATCH = 4096
SEQ = 200

NC = 2
NS = 16
NW = NC * NS
B = BATCH * SEQ
CHUNK = 128
B_PER_W = B // NW
N_CHUNKS = B_PER_W // CHUNK
NBUF = 8
N_OUTER = N_CHUNKS // NBUF


def _make_gather():
    mesh = plsc.VectorSubcoreMesh(core_axis_name="c", subcore_axis_name="s")

    @functools.partial(
        pl.kernel,
        out_type=jax.ShapeDtypeStruct((B, EMBEDDING_DIM), jnp.float32),
        mesh=mesh,
        scratch_types=(
            [pltpu.VMEM((N_CHUNKS, CHUNK), jnp.int32),
             pltpu.VMEM((NBUF, CHUNK, EMBEDDING_DIM), jnp.float32)]
            + [pltpu.SemaphoreType.DMA] * (2 * NBUF)
        ),
        compiler_params=pltpu.CompilerParams(use_tc_tiling_on_sc=False),
    )
    def gather_kernel(idx_hbm, table_hbm, out_hbm, idx_v, rows_v, *sems):
        gsems, osems = sems[:NBUF], sems[NBUF:]
        wid = lax.axis_index("s") * NC + lax.axis_index("c")
        base = wid * B_PER_W
        pltpu.sync_copy(idx_hbm.at[wid], idx_v)

        def start_gather(j, s):
            pltpu.async_copy(table_hbm.at[idx_v.at[j]], rows_v.at[s], gsems[s])

        def wait_gather(j, s):
            pltpu.make_async_copy(
                table_hbm.at[idx_v.at[j]], rows_v.at[s], gsems[s]).wait()

        def start_write(j, s):
            pltpu.async_copy(
                rows_v.at[s], out_hbm.at[pl.ds(base + j * CHUNK, CHUNK)],
                osems[s])

        def wait_write(j, s):
            pltpu.make_async_copy(
                rows_v.at[s], out_hbm.at[pl.ds(base + j * CHUNK, CHUNK)],
                osems[s]).wait()

        for b in range(NBUF - 1):
            start_gather(b, b)

        for b in range(NBUF):
            g = b
            wait_gather(g, b)
            start_write(g, b)
            n = g + NBUF - 1
            sn = n % NBUF
            if n >= NBUF:
                wait_write(n - NBUF, sn)
            start_gather(n, sn)

        def body(k, carry):
            g0 = k * NBUF
            for b in range(NBUF):
                g = g0 + b
                wait_gather(g, b)
                start_write(g, b)
                n = g + NBUF - 1
                sn = (b + NBUF - 1) % NBUF
                wait_write(n - NBUF, sn)
                start_gather(n, sn)
            return carry

        lax.fori_loop(1, N_OUTER - 1, body, 0)

        g0 = (N_OUTER - 1) * NBUF
        for b in range(NBUF):
            g = g0 + b
            wait_gather(g, b)
            start_write(g, b)
            n = g + NBUF - 1
            if n < N_CHUNKS:
                sn = n % NBUF
                wait_write(n - NBUF, sn)
                start_gather(n, sn)

        for b in range(NBUF):
            wait_write(g0 + b, b)

    return gather_kernel


_gather = _make_gather()


def kernel(input_ids, weight):
    idx = input_ids.astype(jnp.int32).reshape(NW, N_CHUNKS, CHUNK)
    out = _gather(idx, weight)
    return out.reshape(BATCH, SEQ, EMBEDDING_DIM)
